# baseline (speedup 1.0000x reference)
import functools
import math

import jax
import jax.numpy as jnp
from jax import lax
from jax.experimental import pallas as pl
from jax.experimental.pallas import tpu as pltpu
from jax.experimental.pallas import tpu_sc as plsc

_NC = 2
_NS = 16
_NW = _NC * _NS
_LANES = 16
_NH = 4


def _build_sc_lookup(seq, batch, d):
    ppw = seq // _NW
    hp = ppw // _NH
    scale = float(math.sqrt(d))
    mesh = plsc.VectorSubcoreMesh(core_axis_name="c", subcore_axis_name="s")

    @functools.partial(
        pl.kernel,
        mesh=mesh,
        out_type=jax.ShapeDtypeStruct((seq, batch, d), jnp.float32),
        scratch_types=(
            [pltpu.VMEM((batch, ppw), jnp.int32)]
            + [pltpu.VMEM((batch, ppw, d), jnp.float32)]
            + [pltpu.VMEM((hp, batch, d), jnp.float32) for _ in range(2)]
            + [pltpu.VMEM((ppw, d), jnp.float32)]
            + [pltpu.SemaphoreType.DMA for _ in range(4)]
        ),
    )
    def sc_lookup(table_hbm, xt_hbm, pos_hbm, out_hbm, idxb_v, *bufs):
        gbuf = bufs[0]
        obuf = bufs[1:3]
        pos_v = bufs[3]
        gsem, psem, os0, os1 = bufs[4:]
        osem = (os0, os1)

        wid = lax.axis_index("s") * _NC + lax.axis_index("c")
        base = wid * ppw

        pltpu.sync_copy(xt_hbm.at[:, pl.ds(base, ppw)], idxb_v)
        gcopies = [
            pltpu.async_copy(table_hbm.at[idxb_v.at[b]], gbuf.at[b], gsem)
            for b in range(batch)
        ]
        pos_cp = pltpu.async_copy(pos_hbm.at[pl.ds(base, ppw)], pos_v, psem)
        for cp in gcopies:
            cp.wait()
        pos_cp.wait()

        out_flight = {}
        for h in range(_NH):
            u = h % 2
            if h >= 2:
                out_flight.pop(h - 2).wait()

            @plsc.parallel_loop(0, hp, unroll=8)
            def step(p, u=u, h=h):
                pos_regs = [
                    pos_v[h * hp + p, pl.ds(k * _LANES, _LANES)]
                    for k in range(d // _LANES)
                ]
                for b in range(batch):
                    for k in range(d // _LANES):
                        sl = pl.ds(k * _LANES, _LANES)
                        obuf[u][p, b, sl] = (
                            gbuf[b, h * hp + p, sl] * scale + pos_regs[k]
                        )

            out_flight[h] = pltpu.async_copy(
                obuf[u], out_hbm.at[pl.ds(base + h * hp, hp)], osem[u]
            )
        for h in sorted(out_flight):
            out_flight.pop(h).wait()

    return sc_lookup


def kernel(x, table, pos_emb):
    seq, batch = x.shape
    d = table.shape[1]
    xt = x.T
    pos2 = pos_emb[:seq].reshape(seq, d)
    return _build_sc_lookup(seq, batch, d)(table, xt, pos2)


# speedup vs baseline: 1.0560x; 1.0560x over previous
import functools
import math

import jax
import jax.numpy as jnp
from jax import lax
from jax.experimental import pallas as pl
from jax.experimental.pallas import tpu as pltpu
from jax.experimental.pallas import tpu_sc as plsc

_NC = 2
_NS = 16
_NW = _NC * _NS
_LANES = 16
_NH = 4


def _build_sc_lookup(seq, batch, d):
    ppw = seq // _NW
    hp = ppw // _NH
    scale = float(math.sqrt(d))
    mesh = plsc.VectorSubcoreMesh(core_axis_name="c", subcore_axis_name="s")

    @functools.partial(
        pl.kernel,
        mesh=mesh,
        out_type=jax.ShapeDtypeStruct((seq, batch, d), jnp.float32),
        scratch_types=(
            [pltpu.VMEM((batch, ppw), jnp.int32)]
            + [pltpu.VMEM((batch, ppw, d), jnp.float32)]
            + [pltpu.VMEM((hp, batch, d), jnp.float32) for _ in range(2)]
            + [pltpu.VMEM((ppw, d), jnp.float32)]
            + [pltpu.SemaphoreType.DMA for _ in range(4)]
        ),
    )
    def sc_lookup(table_hbm, xt_hbm, pos_hbm, out_hbm, idxb_v, *bufs):
        gbuf = bufs[0]
        obuf = bufs[1:3]
        pos_v = bufs[3]
        gsem, psem, os0, os1 = bufs[4:]
        osem = (os0, os1)

        wid = lax.axis_index("s") * _NC + lax.axis_index("c")
        base = wid * ppw

        pltpu.sync_copy(xt_hbm.at[:, pl.ds(base, ppw)], idxb_v)
        gcopies = [
            pltpu.async_copy(table_hbm.at[idxb_v.at[b]], gbuf.at[b], gsem)
            for b in range(batch)
        ]
        pos_cp = pltpu.async_copy(pos_hbm.at[pl.ds(base, ppw)], pos_v, psem)
        for cp in gcopies:
            cp.wait()
        pos_cp.wait()

        out_flight = {}
        for h in range(_NH):
            u = h % 2
            if h >= 2:
                out_flight.pop(h - 2).wait()

            @plsc.parallel_loop(0, hp, unroll=4)
            def step(p, u=u, h=h):
                pos_regs = [
                    pos_v[h * hp + p, pl.ds(k * _LANES, _LANES)]
                    for k in range(d // _LANES)
                ]
                for b in range(batch):
                    for k in range(d // _LANES):
                        sl = pl.ds(k * _LANES, _LANES)
                        obuf[u][p, b, sl] = (
                            gbuf[b, h * hp + p, sl] * scale + pos_regs[k]
                        )

            out_flight[h] = pltpu.async_copy(
                obuf[u], out_hbm.at[pl.ds(base + h * hp, hp)], osem[u]
            )
        for h in sorted(out_flight):
            out_flight.pop(h).wait()

    return sc_lookup


def kernel(x, table, pos_emb):
    seq, batch = x.shape
    d = table.shape[1]
    xt = x.T
    pos2 = pos_emb[:seq].reshape(seq, d)
    return _build_sc_lookup(seq, batch, d)(table, xt, pos2)
